# baseline (speedup 1.0000x reference)
import jax
import jax.numpy as jnp
from jax.experimental import pallas as pl
from jax.experimental.pallas import tpu as pltpu

_ROWS = 128
_COLS = 100000
_SUP_STRIDE = 200
_SUP_LIMIT = 6400
_TAIL_CHUNKS = 8
_CHUNK_ROWS = _ROWS // _TAIL_CHUNKS


def _body(x_hbm, o_hbm, head_vmem, sem_in, sem_out, sem_tail):
    full_cp = pltpu.make_async_copy(x_hbm, o_hbm, sem_tail.at[0])
    full_cp.start()
    head_in = pltpu.make_async_copy(
        x_hbm.at[:, pl.ds(0, _SUP_LIMIT)], head_vmem, sem_in
    )
    head_in.start()
    head_in.wait()
    neg = jnp.full((_ROWS, 1), -jnp.inf, jnp.float32)
    for c in range(0, _SUP_LIMIT, _SUP_STRIDE):
        head_vmem[:, c : c + 1] = neg
    full_cp.wait()
    head_out = pltpu.make_async_copy(
        head_vmem, o_hbm.at[:, pl.ds(0, _SUP_LIMIT)], sem_out
    )
    head_out.start()
    head_out.wait()


def kernel(scores):
    return pl.pallas_call(
        _body,
        in_specs=[pl.BlockSpec(memory_space=pl.MemorySpace.ANY)],
        out_specs=pl.BlockSpec(memory_space=pl.MemorySpace.ANY),
        out_shape=jax.ShapeDtypeStruct((_ROWS, _COLS), scores.dtype),
        scratch_shapes=[
            pltpu.MemorySpace.VMEM(((_ROWS, _SUP_LIMIT)), jnp.float32),
            pltpu.SemaphoreType.DMA,
            pltpu.SemaphoreType.DMA,
            pltpu.SemaphoreType.DMA((_TAIL_CHUNKS,)),
        ],
    )(scores)


# speedup vs baseline: 13.4014x; 13.4014x over previous
import jax
import jax.numpy as jnp
from jax.experimental import pallas as pl
from jax.experimental.pallas import tpu as pltpu

_ROWS = 128
_COLS = 100000
_SUP_STRIDE = 200
_SUP_LIMIT = 6400
_N_CHUNKS = 16
_CHUNK_ROWS = _ROWS // _N_CHUNKS
_N_BUFS = 6


def _chunk_slice(x_hbm, i):
    return x_hbm.at[pl.ds(i * _CHUNK_ROWS, _CHUNK_ROWS), :]


def _body(x_hbm, o_hbm, bufs, sem_in, sem_out):
    def start_in(i):
        pltpu.make_async_copy(
            _chunk_slice(x_hbm, i), bufs.at[i % _N_BUFS], sem_in.at[i]
        ).start()

    for j in range(_N_BUFS):
        start_in(j)
    neg = jnp.full((_CHUNK_ROWS, 1), -jnp.inf, jnp.float32)
    for i in range(_N_CHUNKS):
        b = i % _N_BUFS
        pltpu.make_async_copy(
            _chunk_slice(x_hbm, i), bufs.at[b], sem_in.at[i]
        ).wait()
        for c in range(0, _SUP_LIMIT, _SUP_STRIDE):
            bufs[b, :, c : c + 1] = neg
        pltpu.make_async_copy(
            bufs.at[b], _chunk_slice(o_hbm, i), sem_out.at[i]
        ).start()
        nxt = i + _N_BUFS
        if nxt < _N_CHUNKS:
            pltpu.make_async_copy(
                bufs.at[b], _chunk_slice(o_hbm, i), sem_out.at[i]
            ).wait()
            start_in(nxt)
    for i in range(_N_CHUNKS - _N_BUFS, _N_CHUNKS):
        pltpu.make_async_copy(
            bufs.at[i % _N_BUFS], _chunk_slice(o_hbm, i), sem_out.at[i]
        ).wait()


def kernel(scores):
    return pl.pallas_call(
        _body,
        in_specs=[pl.BlockSpec(memory_space=pl.MemorySpace.ANY)],
        out_specs=pl.BlockSpec(memory_space=pl.MemorySpace.ANY),
        out_shape=jax.ShapeDtypeStruct((_ROWS, _COLS), scores.dtype),
        scratch_shapes=[
            pltpu.MemorySpace.VMEM((_N_BUFS, _CHUNK_ROWS, _COLS), jnp.float32),
            pltpu.SemaphoreType.DMA((_N_CHUNKS,)),
            pltpu.SemaphoreType.DMA((_N_CHUNKS,)),
        ],
    )(scores)
